# baseline (speedup 1.0000x reference)
import functools

import jax
import jax.numpy as jnp
from jax import lax
from jax.experimental import pallas as pl
from jax.experimental.pallas import tpu as pltpu
from jax.experimental.pallas import tpu_sc as plsc

HIDDEN = 64
NC, NS = 2, 16
NW = NC * NS
GSZ = 128


def _sc_fused(table, comb, idx2d, c2d, gamma, beta, rows, eps=1e-5):
    ng_total = idx2d.shape[0]
    ng = ng_total // NW
    NBUF = 4
    LOOK = 2
    lpc = GSZ // 2
    rpw = ng * GSZ

    mesh = plsc.VectorSubcoreMesh(core_axis_name="c", subcore_axis_name="s")

    @functools.partial(
        pl.kernel,
        mesh=mesh,
        compiler_params=pltpu.CompilerParams(
            use_tc_tiling_on_sc=False, needs_layout_passes=False
        ),
        out_type=jax.ShapeDtypeStruct((rows, HIDDEN), jnp.float32),
        scratch_types=[
            pltpu.VMEM((ng, GSZ), jnp.int32),
            pltpu.VMEM((ng, lpc), jnp.int32),
            pltpu.VMEM((NBUF, GSZ, HIDDEN), jnp.float32),
            pltpu.VMEM((NBUF, lpc, 2 * HIDDEN), jnp.float32),
            pltpu.VMEM((HIDDEN,), jnp.float32),
            pltpu.VMEM((HIDDEN,), jnp.float32),
            pltpu.SemaphoreType.DMA((NBUF,)),
            pltpu.SemaphoreType.DMA((NBUF,)),
            pltpu.SemaphoreType.DMA((NBUF,)),
        ],
    )
    def k(table_hbm, comb_hbm, idx_hbm, c2_hbm, gam_hbm, bet_hbm, out_hbm,
          idx_v, c2_v, g_v, c_v, gam_v, bet_v, gsem, csem, osem):
        wid = lax.axis_index("s") * NC + lax.axis_index("c")
        pltpu.sync_copy(idx_hbm.at[pl.ds(wid * ng, ng)], idx_v)
        pltpu.sync_copy(c2_hbm.at[pl.ds(wid * ng, ng)], c2_v)
        pltpu.sync_copy(gam_hbm, gam_v)
        pltpu.sync_copy(bet_hbm, bet_v)

        gq = [gam_v[pl.ds(16 * q, 16)] for q in range(4)]
        bq = [bet_v[pl.ds(16 * q, 16)] for q in range(4)]
        iota16 = lax.iota(jnp.int32, 16)
        perms = [iota16 ^ (1 << p) for p in range(4)]
        magic = jnp.zeros((16,), jnp.int32) + 0x5F3759DF
        inv_h = 1.0 / HIDDEN

        def fire(j, b):
            pltpu.async_copy(table_hbm.at[idx_v.at[j]], g_v.at[b], gsem.at[b])
            pltpu.async_copy(comb_hbm.at[c2_v.at[j]], c_v.at[b], csem.at[b])

        def wait_gathers(b):
            pltpu.make_async_copy(
                table_hbm.at[pl.ds(0, GSZ)], g_v.at[b], gsem.at[b]
            ).wait()
            pltpu.make_async_copy(
                comb_hbm.at[pl.ds(0, lpc)], c_v.at[b], csem.at[b]
            ).wait()

        def wait_outcopy(b):
            pltpu.make_async_copy(
                g_v.at[b], out_hbm.at[pl.ds(0, GSZ)], osem.at[b]
            ).wait()

        for j0 in range(LOOK):
            fire(j0, j0)

        def allsum(v):
            for p in perms:
                v = v + v[p]
            return v

        def row_ln(b, i):
            x = [
                g_v[b, i, pl.ds(16 * q, 16)]
                + c_v[b, i // 2, pl.ds((i % 2) * HIDDEN + 16 * q, 16)]
                for q in range(4)
            ]
            tot = allsum((x[0] + x[1]) + (x[2] + x[3]))
            mean = tot * inv_h
            d = [xq - mean for xq in x]
            ssq = (d[0] * d[0] + d[1] * d[1]) + (d[2] * d[2] + d[3] * d[3])
            var = allsum(ssq) * inv_h
            v = var + eps
            y = plsc.bitcast(magic - (plsc.bitcast(v, jnp.int32) >> 1),
                             jnp.float32)
            h = v * 0.5
            for _ in range(3):
                y = y * (1.5 - h * y * y)
            for q in range(4):
                g_v[b, i, pl.ds(16 * q, 16)] = d[q] * (y * gq[q]) + bq[q]

        def body(j, carry):
            b = lax.rem(j, NBUF)
            wait_gathers(b)
            jn = j + LOOK
            bn = lax.rem(jn, NBUF)

            @pl.when(jn < ng)
            def _():
                @pl.when(j >= NBUF - LOOK)
                def _():
                    wait_outcopy(bn)

                fire(jn, bn)

            def group(g, cc):
                i0 = g * 16
                for di in range(16):
                    row_ln(b, i0 + di)
                return cc

            lax.fori_loop(0, GSZ // 16, group, 0)

            pltpu.async_copy(
                g_v.at[b],
                out_hbm.at[pl.ds(wid * rpw + j * GSZ, GSZ)],
                osem.at[b],
            )
            return carry

        lax.fori_loop(0, ng, body, 0)
        for b in range(NBUF):
            wait_outcopy(b)

    return k(table, comb, idx2d, c2d, gamma, beta)


def kernel(token_ids, type_ids, token_table, type_table, pos_table, ln_gamma, ln_beta):
    B, S = token_ids.shape
    rows = B * S
    lines = rows // 2
    hs = S // 2
    idx2d = token_ids.reshape(rows // GSZ, GSZ).astype(jnp.int32)

    ntypes = type_table.shape[0]
    pos_pair = pos_table[:S].reshape(1, hs, 2 * HIDDEN)
    ta = jnp.repeat(type_table, ntypes, axis=0)
    tb = jnp.tile(type_table, (ntypes, 1))
    tcat = jnp.concatenate([ta, tb], axis=1)
    comb = (tcat[:, None, :] + pos_pair).reshape(ntypes * ntypes * hs, 2 * HIDDEN)

    t2 = type_ids.astype(jnp.int32).reshape(lines, 2)
    pcode = t2[:, 0] * ntypes + t2[:, 1]
    ppos = jax.lax.broadcasted_iota(jnp.int32, (lines,), 0) % hs
    c2d = (pcode * hs + ppos).reshape(rows // GSZ, GSZ // 2)

    out = _sc_fused(token_table, comb, idx2d, c2d, ln_gamma, ln_beta, rows)
    return out.reshape(B, S, HIDDEN)


# speedup vs baseline: 1.6143x; 1.6143x over previous
import functools

import jax
import jax.numpy as jnp
from jax import lax
from jax.experimental import pallas as pl
from jax.experimental.pallas import tpu as pltpu
from jax.experimental.pallas import tpu_sc as plsc

HIDDEN = 64
NC, NS = 2, 16
NW = NC * NS
GSZ = 128


def _sc_gather_sum(table, comb, idx2d, c2d, rows):
    ng_total = idx2d.shape[0]
    ng = ng_total // NW
    NBUF = 4
    LOOK = 2
    lpc = GSZ // 2
    rpw = ng * GSZ

    mesh = plsc.VectorSubcoreMesh(core_axis_name="c", subcore_axis_name="s")

    @functools.partial(
        pl.kernel,
        mesh=mesh,
        compiler_params=pltpu.CompilerParams(
            use_tc_tiling_on_sc=False, needs_layout_passes=False
        ),
        out_type=jax.ShapeDtypeStruct((rows, 2 * HIDDEN), jnp.float32),
        scratch_types=[
            pltpu.VMEM((ng, GSZ), jnp.int32),
            pltpu.VMEM((ng, lpc), jnp.int32),
            pltpu.VMEM((NBUF, GSZ, HIDDEN), jnp.float32),
            pltpu.VMEM((NBUF, lpc, 2 * HIDDEN), jnp.float32),
            pltpu.SemaphoreType.DMA((NBUF,)),
            pltpu.SemaphoreType.DMA((NBUF,)),
            pltpu.SemaphoreType.DMA((NBUF,)),
        ],
    )
    def k(table_hbm, comb_hbm, idx_hbm, c2_hbm, out_hbm,
          idx_v, c2_v, g_v, c_v, gsem, csem, osem):
        wid = lax.axis_index("s") * NC + lax.axis_index("c")
        pltpu.sync_copy(idx_hbm.at[pl.ds(wid * ng, ng)], idx_v)
        pltpu.sync_copy(c2_hbm.at[pl.ds(wid * ng, ng)], c2_v)

        def fire(j, b):
            pltpu.async_copy(table_hbm.at[idx_v.at[j]], g_v.at[b], gsem.at[b])
            pltpu.async_copy(comb_hbm.at[c2_v.at[j]], c_v.at[b], csem.at[b])

        def wait_gathers(b):
            pltpu.make_async_copy(
                table_hbm.at[pl.ds(0, GSZ)], g_v.at[b], gsem.at[b]
            ).wait()
            pltpu.make_async_copy(
                comb_hbm.at[pl.ds(0, lpc)], c_v.at[b], csem.at[b]
            ).wait()

        def wait_outcopy(b):
            pltpu.make_async_copy(
                g_v.at[b],
                out_hbm.at[pl.ds(0, GSZ), pl.ds(0, HIDDEN)],
                osem.at[b],
            ).wait()

        for j0 in range(LOOK):
            fire(j0, j0)

        def body(j, carry):
            b = lax.rem(j, NBUF)
            wait_gathers(b)
            jn = j + LOOK
            bn = lax.rem(jn, NBUF)

            @pl.when(jn < ng)
            def _():
                @pl.when(j >= NBUF - LOOK)
                def _():
                    wait_outcopy(bn)

                fire(jn, bn)

            for i in range(GSZ):
                for q in range(4):
                    sl = pl.ds(16 * q, 16)
                    cl = pl.ds((i % 2) * HIDDEN + 16 * q, 16)
                    g_v[b, i, sl] = g_v[b, i, sl] + c_v[b, i // 2, cl]

            pltpu.async_copy(
                g_v.at[b],
                out_hbm.at[pl.ds(wid * rpw + j * GSZ, GSZ), pl.ds(0, HIDDEN)],
                osem.at[b],
            )
            return carry

        lax.fori_loop(0, ng, body, 0)
        for b in range(NBUF):
            wait_outcopy(b)

    return k(table, comb, idx2d, c2d)


def _tc_ln(embp, gamma, beta, rows):
    R = 6400

    def body(x_ref, g_ref, b_ref, o_ref):
        xr = x_ref[...]
        lane = lax.broadcasted_iota(jnp.int32, (R, 2 * HIDDEN), 1)
        x = jnp.where(lane < HIDDEN, xr, 0.0)
        i0 = lax.broadcasted_iota(jnp.int32, (2 * HIDDEN, 2 * HIDDEN), 0)
        i1 = lax.broadcasted_iota(jnp.int32, (2 * HIDDEN, 2 * HIDDEN), 1)
        m = jnp.where((i0 // HIDDEN) == (i1 // HIDDEN), 1.0 / HIDDEN, 0.0)
        mean = jnp.dot(x, m, preferred_element_type=jnp.float32)
        msq = jnp.dot(x * x, m, preferred_element_type=jnp.float32)
        var = msq - mean * mean
        y = (x - mean) * lax.rsqrt(var + 1e-5) * g_ref[...] + b_ref[...]
        o_ref[...] = y[:, :HIDDEN]

    return pl.pallas_call(
        body,
        grid=(rows // R,),
        in_specs=[
            pl.BlockSpec((R, 2 * HIDDEN), lambda i: (i, 0)),
            pl.BlockSpec((1, 2 * HIDDEN), lambda i: (0, 0)),
            pl.BlockSpec((1, 2 * HIDDEN), lambda i: (0, 0)),
        ],
        out_specs=pl.BlockSpec((R, HIDDEN), lambda i: (i, 0)),
        out_shape=jax.ShapeDtypeStruct((rows, HIDDEN), jnp.float32),
    )(embp, jnp.tile(gamma, 2).reshape(1, -1), jnp.tile(beta, 2).reshape(1, -1))


def kernel(token_ids, type_ids, token_table, type_table, pos_table, ln_gamma, ln_beta):
    B, S = token_ids.shape
    rows = B * S
    lines = rows // 2
    hs = S // 2
    idx2d = token_ids.reshape(rows // GSZ, GSZ).astype(jnp.int32)

    ntypes = type_table.shape[0]
    pos_pair = pos_table[:S].reshape(1, hs, 2 * HIDDEN)
    ta = jnp.repeat(type_table, ntypes, axis=0)
    tb = jnp.tile(type_table, (ntypes, 1))
    tcat = jnp.concatenate([ta, tb], axis=1)
    comb = (tcat[:, None, :] + pos_pair).reshape(ntypes * ntypes * hs, 2 * HIDDEN)

    t2 = type_ids.astype(jnp.int32).reshape(lines, 2)
    pcode = t2[:, 0] * ntypes + t2[:, 1]
    ppos = jax.lax.broadcasted_iota(jnp.int32, (lines,), 0) % hs
    c2d = (pcode * hs + ppos).reshape(rows // GSZ, GSZ // 2)

    embp = _sc_gather_sum(token_table, comb, idx2d, c2d, rows)
    out = _tc_ln(embp, ln_gamma, ln_beta, rows)
    return out.reshape(B, S, HIDDEN)
